# baseline (speedup 1.0000x reference)
import functools

import jax
import jax.numpy as jnp
from jax import lax
from jax.experimental import pallas as pl
from jax.experimental.pallas import tpu as pltpu
from jax.experimental.pallas import tpu_sc as plsc

NUM_EMB = 1000000
EMBEDDING_DIM = 64
RANK = 8
LORA_SCALING = 16.0 / 8.0

NUM_CORES = 2
NUM_SUBCORES = 16
NUM_WORKERS = NUM_CORES * NUM_SUBCORES
BCHUNK = 128
NBUF = 2
LANES = 16
NGROUPS = BCHUNK // LANES

TC_IBLK = 4096
PACKED_ROWS = 524288


def _full16(v):
    return jnp.full((LANES,), v, jnp.int32)


def _fuse_table(wt, at, sbt):
    grid = PACKED_ROWS // TC_IBLK
    hblk = PACKED_ROWS // TC_IBLK
    lastblk = NUM_EMB // TC_IBLK

    def _hi_map(g):
        return (0, jnp.minimum(g + hblk, lastblk))

    def body(wt0_ref, at0_ref, wt1_ref, at1_ref, sbt_ref, out_ref):
        sb = sbt_ref[...]
        st0 = wt0_ref[...] + jax.lax.dot_general(
            sb, at0_ref[...], (((1,), (0,)), ((), ())),
            preferred_element_type=jnp.float32)
        st1 = wt1_ref[...] + jax.lax.dot_general(
            sb, at1_ref[...], (((1,), (0,)), ((), ())),
            preferred_element_type=jnp.float32)
        out_ref[:, 0:EMBEDDING_DIM] = st0.T
        out_ref[:, EMBEDDING_DIM:128] = st1.T

    return pl.pallas_call(
        body,
        grid=(grid,),
        in_specs=[
            pl.BlockSpec((EMBEDDING_DIM, TC_IBLK), lambda g: (0, g)),
            pl.BlockSpec((RANK, TC_IBLK), lambda g: (0, g)),
            pl.BlockSpec((EMBEDDING_DIM, TC_IBLK), _hi_map),
            pl.BlockSpec((RANK, TC_IBLK), _hi_map),
            pl.BlockSpec((EMBEDDING_DIM, RANK), lambda g: (0, 0)),
        ],
        out_specs=pl.BlockSpec((TC_IBLK, 128), lambda g: (g, 0)),
        out_shape=jax.ShapeDtypeStruct((PACKED_ROWS, 128), jnp.float32),
    )(wt, at, wt, at, sbt)


@functools.partial(jax.jit, static_argnames=("n_s",))
def _gather(xt, s2, n_s):
    nb = NUM_WORKERS * BCHUNK

    def body(x_hbm, s2_hbm, out_hbm, xt_v, i0, i1,
             w0, w1, t0, t1,
             wsem0, wsem1, osem0, osem1):
        wid = lax.axis_index("s") * NUM_CORES + lax.axis_index("c")
        b_base = wid * BCHUNK

        ibufs = (i0, i1)
        w_bufs = (w0, w1)
        out_bufs = (t0, t1)
        wsems = (wsem0, wsem1)
        osems = (osem0, osem1)

        pltpu.sync_copy(x_hbm.at[:, pl.ds(b_base, BCHUNK)], xt_v)

        def fill_ibuf(ib, s):
            for g in range(NGROUPS):
                xv = xt_v[s, pl.ds(g * LANES, LANES)]
                ib[pl.ds(g * LANES, LANES)] = xv & jnp.int32(PACKED_ROWS - 1)

        for b in range(NBUF):
            fill_ibuf(ibufs[b], b)
            pltpu.make_async_copy(s2_hbm.at[ibufs[b]], w_bufs[b], wsems[b]).start()

        @pl.loop(0, n_s, step=NBUF)
        def s_loop(s0):
            for b in range(NBUF):
                s = s0 + b
                pltpu.make_async_copy(s2_hbm.at[ibufs[b]], w_bufs[b], wsems[b]).wait()

                @pl.when(s0 > 0)
                def _():
                    pltpu.make_async_copy(
                        out_bufs[b],
                        out_hbm.at[s - NBUF, :, pl.ds(b_base, BCHUNK)],
                        osems[b]).wait()

                w_b, out_b = w_bufs[b], out_bufs[b]

                @plsc.parallel_loop(0, NGROUPS, unroll=8)
                def tr_loop(g):
                    riota = _full16(g * LANES) + lax.iota(jnp.int32, LANES)
                    xv = xt_v[s, pl.ds(g * LANES, LANES)]
                    off = lax.shift_right_logical(xv, 13) & jnp.int32(EMBEDDING_DIM)
                    for d0 in range(0, EMBEDDING_DIM, 8):
                        cols = [plsc.load_gather(w_b, [riota, off + (d0 + k)])
                                for k in range(8)]
                        for k in range(8):
                            out_b[d0 + k, pl.ds(g * LANES, LANES)] = cols[k]

                pltpu.make_async_copy(
                    out_b, out_hbm.at[s, :, pl.ds(b_base, BCHUNK)],
                    osems[b]).start()

                @pl.when(s + NBUF < n_s)
                def _():
                    fill_ibuf(ibufs[b], s + NBUF)
                    pltpu.make_async_copy(
                        s2_hbm.at[ibufs[b]], w_bufs[b], wsems[b]).start()

        for b in range(NBUF):
            s = n_s - NBUF + b
            pltpu.make_async_copy(
                out_bufs[b], out_hbm.at[s, :, pl.ds(b_base, BCHUNK)],
                osems[b]).wait()

    run = pl.kernel(
        body,
        out_type=jax.ShapeDtypeStruct((n_s, EMBEDDING_DIM, nb), jnp.float32),
        mesh=plsc.VectorSubcoreMesh(core_axis_name="c", subcore_axis_name="s"),
        compiler_params=pltpu.CompilerParams(
            needs_layout_passes=False, use_tc_tiling_on_sc=False),
        scratch_types=(
            [pltpu.VMEM((n_s, BCHUNK), jnp.int32)]
            + [pltpu.VMEM((BCHUNK,), jnp.int32)] * NBUF
            + [pltpu.VMEM((BCHUNK, 128), jnp.float32)] * NBUF
            + [pltpu.VMEM((EMBEDDING_DIM, BCHUNK), jnp.float32)] * NBUF
            + [pltpu.SemaphoreType.DMA] * (2 * NBUF)
        ),
    )
    return run(xt, s2)


def kernel(x, weight, lora_A, lora_B):
    nb, n_s = x.shape
    assert nb == NUM_WORKERS * BCHUNK
    wt = weight.T
    at = lora_A.T
    sbt = (lora_B * LORA_SCALING).T
    s2 = _fuse_table(wt, at, sbt)

    xt = x.T.astype(jnp.int32)
    out_t = _gather(xt, s2, n_s)
    return jnp.transpose(out_t, (2, 0, 1))


# speedup vs baseline: 1.0528x; 1.0528x over previous
import functools

import jax
import jax.numpy as jnp
from jax import lax
from jax.experimental import pallas as pl
from jax.experimental.pallas import tpu as pltpu
from jax.experimental.pallas import tpu_sc as plsc

NUM_EMB = 1000000
EMBEDDING_DIM = 64
RANK = 8
LORA_SCALING = 16.0 / 8.0

NUM_CORES = 2
NUM_SUBCORES = 16
NUM_WORKERS = NUM_CORES * NUM_SUBCORES
BCHUNK = 128
NBUF = 4
LANES = 16
NGROUPS = BCHUNK // LANES

TC_IBLK = 4096
PACKED_ROWS = 524288


def _full16(v):
    return jnp.full((LANES,), v, jnp.int32)


def _fuse_table(wt, at, sbt):
    grid = PACKED_ROWS // TC_IBLK
    hblk = PACKED_ROWS // TC_IBLK
    lastblk = NUM_EMB // TC_IBLK

    def _hi_map(g):
        return (0, jnp.minimum(g + hblk, lastblk))

    def body(wt0_ref, at0_ref, wt1_ref, at1_ref, sbt_ref, out_ref):
        sb = sbt_ref[...]
        st0 = wt0_ref[...] + jax.lax.dot_general(
            sb, at0_ref[...], (((1,), (0,)), ((), ())),
            preferred_element_type=jnp.float32)
        st1 = wt1_ref[...] + jax.lax.dot_general(
            sb, at1_ref[...], (((1,), (0,)), ((), ())),
            preferred_element_type=jnp.float32)
        out_ref[:, 0:EMBEDDING_DIM] = st0.T
        out_ref[:, EMBEDDING_DIM:128] = st1.T

    return pl.pallas_call(
        body,
        grid=(grid,),
        in_specs=[
            pl.BlockSpec((EMBEDDING_DIM, TC_IBLK), lambda g: (0, g)),
            pl.BlockSpec((RANK, TC_IBLK), lambda g: (0, g)),
            pl.BlockSpec((EMBEDDING_DIM, TC_IBLK), _hi_map),
            pl.BlockSpec((RANK, TC_IBLK), _hi_map),
            pl.BlockSpec((EMBEDDING_DIM, RANK), lambda g: (0, 0)),
        ],
        out_specs=pl.BlockSpec((TC_IBLK, 128), lambda g: (g, 0)),
        out_shape=jax.ShapeDtypeStruct((PACKED_ROWS, 128), jnp.float32),
    )(wt, at, wt, at, sbt)


@functools.partial(jax.jit, static_argnames=("n_s",))
def _gather(xt, s2, n_s):
    nb = NUM_WORKERS * BCHUNK

    def body(x_hbm, s2_hbm, out_hbm, xt_v, i0, i1, i2, i3,
             w0, w1, w2, w3, t0, t1, t2, t3,
             wsem0, wsem1, wsem2, wsem3, osem0, osem1, osem2, osem3):
        wid = lax.axis_index("s") * NUM_CORES + lax.axis_index("c")
        b_base = wid * BCHUNK

        ibufs = (i0, i1, i2, i3)
        w_bufs = (w0, w1, w2, w3)
        out_bufs = (t0, t1, t2, t3)
        wsems = (wsem0, wsem1, wsem2, wsem3)
        osems = (osem0, osem1, osem2, osem3)

        pltpu.sync_copy(x_hbm.at[:, pl.ds(b_base, BCHUNK)], xt_v)

        def fill_ibuf(ib, s):
            for g in range(NGROUPS):
                xv = xt_v[s, pl.ds(g * LANES, LANES)]
                ib[pl.ds(g * LANES, LANES)] = xv & jnp.int32(PACKED_ROWS - 1)

        for b in range(NBUF):
            fill_ibuf(ibufs[b], b)
            pltpu.make_async_copy(s2_hbm.at[ibufs[b]], w_bufs[b], wsems[b]).start()

        @pl.loop(0, n_s, step=NBUF)
        def s_loop(s0):
            for b in range(NBUF):
                s = s0 + b
                pltpu.make_async_copy(s2_hbm.at[ibufs[b]], w_bufs[b], wsems[b]).wait()

                @pl.when(s0 > 0)
                def _():
                    pltpu.make_async_copy(
                        out_bufs[b],
                        out_hbm.at[s - NBUF, :, pl.ds(b_base, BCHUNK)],
                        osems[b]).wait()

                w_b, out_b = w_bufs[b], out_bufs[b]

                @plsc.parallel_loop(0, NGROUPS, unroll=4)
                def tr_loop(g):
                    riota = _full16(g * LANES) + lax.iota(jnp.int32, LANES)
                    xv = xt_v[s, pl.ds(g * LANES, LANES)]
                    off = lax.shift_right_logical(xv, 13) & jnp.int32(EMBEDDING_DIM)
                    for d0 in range(0, EMBEDDING_DIM, 8):
                        cols = [plsc.load_gather(w_b, [riota, off + (d0 + k)])
                                for k in range(8)]
                        for k in range(8):
                            out_b[d0 + k, pl.ds(g * LANES, LANES)] = cols[k]

                pltpu.make_async_copy(
                    out_b, out_hbm.at[s, :, pl.ds(b_base, BCHUNK)],
                    osems[b]).start()

                @pl.when(s + NBUF < n_s)
                def _():
                    fill_ibuf(ibufs[b], s + NBUF)
                    pltpu.make_async_copy(
                        s2_hbm.at[ibufs[b]], w_bufs[b], wsems[b]).start()

        for b in range(NBUF):
            s = n_s - NBUF + b
            pltpu.make_async_copy(
                out_bufs[b], out_hbm.at[s, :, pl.ds(b_base, BCHUNK)],
                osems[b]).wait()

    run = pl.kernel(
        body,
        out_type=jax.ShapeDtypeStruct((n_s, EMBEDDING_DIM, nb), jnp.float32),
        mesh=plsc.VectorSubcoreMesh(core_axis_name="c", subcore_axis_name="s"),
        compiler_params=pltpu.CompilerParams(
            needs_layout_passes=False, use_tc_tiling_on_sc=False),
        scratch_types=(
            [pltpu.VMEM((n_s, BCHUNK), jnp.int32)]
            + [pltpu.VMEM((BCHUNK,), jnp.int32)] * NBUF
            + [pltpu.VMEM((BCHUNK, 128), jnp.float32)] * NBUF
            + [pltpu.VMEM((EMBEDDING_DIM, BCHUNK), jnp.float32)] * NBUF
            + [pltpu.SemaphoreType.DMA] * (2 * NBUF)
        ),
    )
    return run(xt, s2)


def kernel(x, weight, lora_A, lora_B):
    nb, n_s = x.shape
    assert nb == NUM_WORKERS * BCHUNK
    wt = weight.T
    at = lora_A.T
    sbt = (lora_B * LORA_SCALING).T
    s2 = _fuse_table(wt, at, sbt)

    xt = x.T.astype(jnp.int32)
    out_t = _gather(xt, s2, n_s)
    return jnp.transpose(out_t, (2, 0, 1))
